# Initial kernel scaffold; baseline (speedup 1.0000x reference)
import functools

import jax
import jax.numpy as jnp
from jax import lax
from jax.experimental import pallas as pl
from jax.experimental.pallas import tpu as pltpu
from jax.experimental.pallas import tpu_sc as plsc

_LANE = 16
_NC, _NS = 2, 16
_NW = _NC * _NS


@functools.lru_cache(maxsize=None)
def _build(batch, seq_len, emb_dim):
    n_tok = batch * seq_len
    rows_per_w = batch // _NW
    cb = 2
    tok_chunk = cb * seq_len
    n_chunks = rows_per_w // cb
    subs = []
    off = 0
    while off < tok_chunk:
        sz = min(128, tok_chunk - off)
        subs.append((off, sz))
        off += sz

    mesh = plsc.VectorSubcoreMesh(core_axis_name="c", subcore_axis_name="s")

    @functools.partial(
        pl.kernel,
        out_type=jax.ShapeDtypeStruct((n_tok, emb_dim), jnp.float32),
        mesh=mesh,
        scratch_types=[
            pltpu.VMEM((tok_chunk,), jnp.int32),
            pltpu.VMEM((tok_chunk, emb_dim), jnp.float32),
            pltpu.VMEM((seq_len, emb_dim), jnp.float32),
            pltpu.SemaphoreType.DMA,
        ],
    )
    def emb_kernel(x_hbm, tok_hbm, pos_hbm, out_hbm, idx_v, rows_v, pos_v, sem):
        wid = lax.axis_index("s") * _NC + lax.axis_index("c")
        base = wid * rows_per_w * seq_len
        pltpu.sync_copy(pos_hbm.at[pl.ds(0, seq_len)], pos_v)

        def chunk_body(c, carry):
            off = base + c * tok_chunk
            pltpu.sync_copy(x_hbm.at[pl.ds(off, tok_chunk)], idx_v)
            copies = [
                pltpu.async_copy(
                    tok_hbm.at[idx_v.at[pl.ds(so, sz)]],
                    rows_v.at[pl.ds(so, sz)],
                    sem,
                )
                for so, sz in subs
            ]
            for cp in copies:
                cp.wait()

            def add_body(t, _):
                for d in range(emb_dim // _LANE):
                    pv = pos_v[t, pl.ds(d * _LANE, _LANE)]
                    for r in range(cb):
                        row = r * seq_len + t
                        rows_v[row, pl.ds(d * _LANE, _LANE)] += pv
                return 0

            lax.fori_loop(0, seq_len, add_body, 0)
            pltpu.sync_copy(rows_v, out_hbm.at[pl.ds(off, tok_chunk)])
            return carry

        lax.fori_loop(0, n_chunks, chunk_body, 0)

    return emb_kernel


@jax.jit
def kernel(x, token_table, pos_table):
    batch, seq_len = x.shape
    emb_dim = token_table.shape[1]
    xf = x.reshape(-1).astype(jnp.int32)
    out = _build(batch, seq_len, emb_dim)(xf, token_table, pos_table)
    return out.reshape(batch, seq_len, emb_dim)


# speedup vs baseline: 3.4586x; 3.4586x over previous
import functools

import jax
import jax.numpy as jnp
from jax import lax
from jax.experimental import pallas as pl
from jax.experimental.pallas import tpu as pltpu
from jax.experimental.pallas import tpu_sc as plsc

_LANE = 16
_NC, _NS = 2, 16
_NW = _NC * _NS


@functools.lru_cache(maxsize=None)
def _build(batch, seq_len, emb_dim):
    n_tok = batch * seq_len
    rows_per_w = batch // _NW
    cb = 2
    tok_chunk = cb * seq_len
    n_chunks = rows_per_w // cb
    subs = []
    off = 0
    while off < tok_chunk:
        sz = min(128, tok_chunk - off)
        subs.append((off, sz))
        off += sz

    mesh = plsc.VectorSubcoreMesh(core_axis_name="c", subcore_axis_name="s")

    @functools.partial(
        pl.kernel,
        out_type=jax.ShapeDtypeStruct((n_tok, emb_dim), jnp.float32),
        mesh=mesh,
        scratch_types=[
            pltpu.VMEM((tok_chunk,), jnp.int32),
            pltpu.VMEM((tok_chunk, emb_dim), jnp.float32),
            pltpu.VMEM((seq_len, emb_dim), jnp.float32),
            pltpu.SemaphoreType.DMA,
        ],
        compiler_params=pltpu.CompilerParams(use_tc_tiling_on_sc=False),
    )
    def emb_kernel(x_hbm, tok_hbm, pos_hbm, out_hbm, idx_v, rows_v, pos_v, sem):
        wid = lax.axis_index("s") * _NC + lax.axis_index("c")
        base = wid * rows_per_w * seq_len
        pltpu.sync_copy(pos_hbm.at[pl.ds(0, seq_len)], pos_v)

        def chunk_body(c, carry):
            off = base + c * tok_chunk
            pltpu.sync_copy(x_hbm.at[pl.ds(off, tok_chunk)], idx_v)
            copies = [
                pltpu.async_copy(
                    tok_hbm.at[idx_v.at[pl.ds(so, sz)]],
                    rows_v.at[pl.ds(so, sz)],
                    sem,
                )
                for so, sz in subs
            ]
            for cp in copies:
                cp.wait()

            def add_body(t, _):
                for d in range(emb_dim // _LANE):
                    pv = pos_v[t, pl.ds(d * _LANE, _LANE)]
                    for r in range(cb):
                        row = r * seq_len + t
                        rows_v[row, pl.ds(d * _LANE, _LANE)] += pv
                return 0

            lax.fori_loop(0, seq_len, add_body, 0)
            pltpu.sync_copy(rows_v, out_hbm.at[pl.ds(off, tok_chunk)])
            return carry

        lax.fori_loop(0, n_chunks, chunk_body, 0)

    return emb_kernel


@jax.jit
def kernel(x, token_table, pos_table):
    batch, seq_len = x.shape
    emb_dim = token_table.shape[1]
    xf = x.reshape(-1).astype(jnp.int32)
    out = _build(batch, seq_len, emb_dim)(xf, token_table, pos_table)
    return out.reshape(batch, seq_len, emb_dim)


# speedup vs baseline: 4.2381x; 1.2254x over previous
import functools

import jax
import jax.numpy as jnp
from jax import lax
from jax.experimental import pallas as pl
from jax.experimental.pallas import tpu as pltpu
from jax.experimental.pallas import tpu_sc as plsc

_LANE = 16
_NC, _NS = 2, 16
_NW = _NC * _NS
_NSLOTS = 4


@functools.lru_cache(maxsize=None)
def _build(batch, seq_len, emb_dim):
    n_tok = batch * seq_len
    tok_per_w = n_tok // _NW
    tok_chunk = seq_len
    n_chunks = tok_per_w // tok_chunk
    n_groups = emb_dim // _LANE
    subs = []
    off = 0
    while off < tok_chunk:
        sz = min(128, tok_chunk - off)
        subs.append((off, sz))
        off += sz

    mesh = plsc.VectorSubcoreMesh(core_axis_name="c", subcore_axis_name="s")

    @functools.partial(
        pl.kernel,
        out_type=jax.ShapeDtypeStruct((n_tok, emb_dim), jnp.float32),
        mesh=mesh,
        scratch_types=[
            pltpu.VMEM((tok_per_w,), jnp.int32),
            pltpu.VMEM((_NSLOTS, tok_chunk, emb_dim), jnp.float32),
            pltpu.VMEM((seq_len, emb_dim), jnp.float32),
        ]
        + [pltpu.SemaphoreType.DMA] * (2 * _NSLOTS),
        compiler_params=pltpu.CompilerParams(use_tc_tiling_on_sc=False),
    )
    def emb_kernel(x_hbm, tok_hbm, pos_hbm, out_hbm, idx_v, rows_v, pos_v,
                   *sems):
        sem_g = sems[:_NSLOTS]
        sem_o = sems[_NSLOTS:]
        wid = lax.axis_index("s") * _NC + lax.axis_index("c")
        base = wid * tok_per_w
        pltpu.sync_copy(pos_hbm.at[pl.ds(0, seq_len)], pos_v)
        pltpu.sync_copy(x_hbm.at[pl.ds(base, tok_per_w)], idx_v)

        def fire_gather(c, slot):
            for so, sz in subs:
                pltpu.async_copy(
                    tok_hbm.at[idx_v.at[pl.ds(c * tok_chunk + so, sz)]],
                    rows_v.at[slot].at[pl.ds(so, sz)],
                    sem_g[slot],
                )

        def wait_gather(c, slot):
            for so, sz in subs:
                pltpu.make_async_copy(
                    tok_hbm.at[idx_v.at[pl.ds(c * tok_chunk + so, sz)]],
                    rows_v.at[slot].at[pl.ds(so, sz)],
                    sem_g[slot],
                ).wait()

        def fire_scatter(c, slot):
            pltpu.async_copy(
                rows_v.at[slot],
                out_hbm.at[pl.ds(base + c * tok_chunk, tok_chunk)],
                sem_o[slot],
            )

        def wait_scatter(c, slot):
            pltpu.make_async_copy(
                rows_v.at[slot],
                out_hbm.at[pl.ds(base + c * tok_chunk, tok_chunk)],
                sem_o[slot],
            ).wait()

        fire_gather(0, 0)
        fire_gather(1, 1)

        def body(q, carry):
            for j in range(_NSLOTS):
                c = q * _NSLOTS + j
                nxt = (j + 2) % _NSLOTS

                @pl.when(c + 2 < n_chunks)
                def _():
                    @pl.when(c >= 2)
                    def _():
                        wait_scatter(c - 2, nxt)

                    fire_gather(c + 2, nxt)

                wait_gather(c, j)

                def add_body(t, _):
                    for d in range(n_groups):
                        sl = pl.ds(d * _LANE, _LANE)
                        rows_v[j, t, sl] += pos_v[t, sl]
                    return 0

                lax.fori_loop(0, tok_chunk, add_body, 0)
                fire_scatter(c, j)
            return carry

        lax.fori_loop(0, n_chunks // _NSLOTS, body, 0)
        wait_scatter(n_chunks - 2, (n_chunks - 2) % _NSLOTS)
        wait_scatter(n_chunks - 1, (n_chunks - 1) % _NSLOTS)

    return emb_kernel


@jax.jit
def kernel(x, token_table, pos_table):
    batch, seq_len = x.shape
    emb_dim = token_table.shape[1]
    xf = x.reshape(-1).astype(jnp.int32)
    out = _build(batch, seq_len, emb_dim)(xf, token_table, pos_table)
    return out.reshape(batch, seq_len, emb_dim)
